# baseline (speedup 1.0000x reference)
import functools

import jax
import jax.numpy as jnp
from jax import lax
from jax.experimental import pallas as pl
from jax.experimental.pallas import tpu as pltpu
from jax.experimental.pallas import tpu_sc as plsc

D_MODEL = 64
SCALE = 8.0

_INFO = plsc.get_sparse_core_info()
NC = _INFO.num_cores
NS = _INFO.num_subcores
NW = NC * NS
LANES = _INFO.num_lanes

TOK = 128
NBUF = 8
HALF = NBUF // 2


def _make_sc_gather(s: int):
  mesh = plsc.VectorSubcoreMesh(core_axis_name="c", subcore_axis_name="s")

  @functools.partial(
      pl.kernel,
      out_type=jax.ShapeDtypeStruct((s, NW, TOK, D_MODEL), jnp.float32),
      mesh=mesh,
      scratch_types=[
          pltpu.VMEM((s, TOK), jnp.int32),
          pltpu.VMEM((NBUF, TOK, D_MODEL), jnp.float32),
          pltpu.SemaphoreType.DMA,
          pltpu.SemaphoreType.DMA,
      ],
      compiler_params=pltpu.CompilerParams(use_tc_tiling_on_sc=False),
  )
  def sc_gather(idx_hbm, table_hbm, out_hbm, idx_v, rows_v, gsem, psem):
    wid = lax.axis_index("s") * NC + lax.axis_index("c")
    pltpu.sync_copy(idx_hbm.at[wid], idx_v)

    def fire_gather(j):
      pltpu.async_copy(table_hbm.at[idx_v.at[j]], rows_v.at[j % NBUF], gsem)

    def wait_one(sem):
      pltpu.make_async_copy(rows_v.at[0], out_hbm.at[0, 0], sem).wait()

    for j in range(HALF):
      fire_gather(j)

    @pl.loop(0, s)
    def _task(j):
      bi = j % NBUF
      wait_one(gsem)

      @pl.loop(0, TOK, unroll=4)
      def _row(r):
        for c in range(D_MODEL // LANES):
          sl = pl.ds(c * LANES, LANES)
          rows_v[bi, r, sl] = rows_v[bi, r, sl] * SCALE

      pltpu.async_copy(rows_v.at[bi], out_hbm.at[j, wid], psem)

      jn = j + HALF

      @pl.when(jn < s)
      def _():
        @pl.when(j >= HALF)
        def _():
          wait_one(psem)
        fire_gather(jn)

    @pl.loop(0, NBUF)
    def _drain(_):
      wait_one(psem)

  return sc_gather


def _tc_relayout_body(in_ref, out_ref):
  x = in_ref[0, 0]
  for g in range(8):
    a = x[:, 8 * g:8 * g + 8]
    bb = x[:, 64 + 8 * g:64 + 8 * g + 8]
    c = jnp.stack([a.T, bb.T], axis=2)
    out_ref[0, g, 0] = c.reshape(8, TOK)


def _make_tc_relayout(s: int):
  return pl.pallas_call(
      _tc_relayout_body,
      grid=(s, NW),
      in_specs=[pl.BlockSpec((1, 1, D_MODEL, TOK),
                             lambda si, bt: (si, bt, 0, 0))],
      out_specs=pl.BlockSpec((1, 8, 1, 8, TOK),
                             lambda si, bt: (si, 0, bt, 0, 0)),
      out_shape=jax.ShapeDtypeStruct((s, 8, NW, 8, TOK), jnp.float32),
  )


def kernel(x, table):
  b, s = x.shape
  idx3 = x.reshape(NW, TOK, s).transpose(0, 2, 1).astype(jnp.int32)
  t = _make_sc_gather(s)(idx3, table)
  t2 = t.reshape(s, NW, D_MODEL, TOK)
  out5 = _make_tc_relayout(s)(t2)
  return jnp.transpose(out5, (2, 4, 0, 1, 3)).reshape(b, s, D_MODEL)


# speedup vs baseline: 4.4165x; 4.4165x over previous
import functools

import jax
import jax.numpy as jnp
import numpy as np
from jax import lax
from jax.experimental import pallas as pl
from jax.experimental.pallas import tpu as pltpu
from jax.experimental.pallas import tpu_sc as plsc

D_MODEL = 64
SCALE = 8.0

_INFO = plsc.get_sparse_core_info()
NC = _INFO.num_cores
NS = _INFO.num_subcores
NW = NC * NS
LANES = _INFO.num_lanes

TOK = 128
NBUF = 8
HALF = NBUF // 2

_M = np.arange(TOK)
_TOKEN_OF_ROW = np.where(_M % 2 == 0, _M // 2, TOK // 2 + (_M - 1) // 2)


def _make_sc_gather(s: int):
  mesh = plsc.VectorSubcoreMesh(core_axis_name="c", subcore_axis_name="s")

  @functools.partial(
      pl.kernel,
      out_type=jax.ShapeDtypeStruct((s, NW, TOK, D_MODEL), jnp.float32),
      mesh=mesh,
      scratch_types=[
          pltpu.VMEM((s, TOK), jnp.int32),
          pltpu.VMEM((NBUF, TOK, D_MODEL), jnp.float32),
          pltpu.SemaphoreType.DMA,
          pltpu.SemaphoreType.DMA,
      ],
      compiler_params=pltpu.CompilerParams(use_tc_tiling_on_sc=False),
  )
  def sc_gather(idx_hbm, table_hbm, out_hbm, idx_v, rows_v, gsem, psem):
    wid = lax.axis_index("s") * NC + lax.axis_index("c")
    pltpu.sync_copy(idx_hbm.at[wid], idx_v)

    def fire_gather(j):
      pltpu.async_copy(table_hbm.at[idx_v.at[j]], rows_v.at[j % NBUF], gsem)

    def wait_one(sem):
      pltpu.make_async_copy(rows_v.at[0], out_hbm.at[0, 0], sem).wait()

    for j in range(HALF):
      fire_gather(j)

    @pl.loop(0, s)
    def _task(j):
      bi = j % NBUF
      wait_one(gsem)

      @pl.loop(0, TOK, unroll=4)
      def _row(r):
        for c in range(D_MODEL // LANES):
          sl = pl.ds(c * LANES, LANES)
          rows_v[bi, r, sl] = rows_v[bi, r, sl] * SCALE

      pltpu.async_copy(rows_v.at[bi], out_hbm.at[j, wid], psem)

      jn = j + HALF

      @pl.when(jn < s)
      def _():
        @pl.when(j >= HALF)
        def _():
          wait_one(psem)
        fire_gather(jn)

    @pl.loop(0, NBUF)
    def _drain(_):
      wait_one(psem)

  return sc_gather


def _tc_relayout_body(in_ref, out_ref):
  x = in_ref[0, 0]
  u = x[:, :D_MODEL]
  v = x[:, D_MODEL:]
  out = jnp.concatenate([u.T, v.T], axis=1)
  out_ref[0, :, 0] = out.reshape(8, 8, TOK)


def _make_tc_relayout(s: int):
  return pl.pallas_call(
      _tc_relayout_body,
      grid=(s, NW),
      in_specs=[pl.BlockSpec((1, 1, D_MODEL, TOK),
                             lambda si, bt: (si, bt, 0, 0))],
      out_specs=pl.BlockSpec((1, 8, 1, 8, TOK),
                             lambda si, bt: (si, 0, bt, 0, 0)),
      out_shape=jax.ShapeDtypeStruct((s, 8, NW, 8, TOK), jnp.float32),
  )


def kernel(x, table):
  b, s = x.shape
  idx3 = x.reshape(NW, TOK, s).transpose(0, 2, 1).astype(jnp.int32)
  idx3 = idx3[:, :, _TOKEN_OF_ROW]
  t = _make_sc_gather(s)(idx3, table)
  t2 = t.reshape(s, NW, D_MODEL, TOK)
  out5 = _make_tc_relayout(s)(t2)
  return jnp.transpose(out5, (2, 4, 0, 1, 3)).reshape(b, s, D_MODEL)


# speedup vs baseline: 10.4976x; 2.3769x over previous
import functools

import jax
import jax.numpy as jnp
import numpy as np
from jax import lax
from jax.experimental import pallas as pl
from jax.experimental.pallas import tpu as pltpu
from jax.experimental.pallas import tpu_sc as plsc

D_MODEL = 64
SCALE = 8.0

_INFO = plsc.get_sparse_core_info()
NC = _INFO.num_cores
NS = _INFO.num_subcores
NW = NC * NS
LANES = _INFO.num_lanes

TOK = 128
NBUF = 8
HALF = NBUF // 2

_M = np.arange(TOK)
_TOKEN_OF_ROW = np.where(_M % 2 == 0, _M // 2, TOK // 2 + (_M - 1) // 2)


def _make_sc_gather(s: int):
  mesh = plsc.VectorSubcoreMesh(core_axis_name="c", subcore_axis_name="s")

  @functools.partial(
      pl.kernel,
      out_type=jax.ShapeDtypeStruct((s, NW, TOK, D_MODEL), jnp.float32),
      mesh=mesh,
      scratch_types=[
          pltpu.VMEM((s, TOK), jnp.int32),
          pltpu.VMEM((NBUF, TOK, D_MODEL), jnp.float32),
          pltpu.SemaphoreType.DMA,
          pltpu.SemaphoreType.DMA,
      ],
      compiler_params=pltpu.CompilerParams(use_tc_tiling_on_sc=False),
  )
  def sc_gather(idx_hbm, table_hbm, out_hbm, idx_v, rows_v, gsem, psem):
    wid = lax.axis_index("s") * NC + lax.axis_index("c")
    pltpu.sync_copy(idx_hbm.at[wid], idx_v)

    def fire_gather(j):
      pltpu.async_copy(table_hbm.at[idx_v.at[j]], rows_v.at[j % NBUF], gsem)

    def wait_one(sem):
      pltpu.make_async_copy(rows_v.at[0], out_hbm.at[0, 0], sem).wait()

    for j in range(HALF):
      fire_gather(j)

    @pl.loop(0, s)
    def _task(j):
      bi = j % NBUF
      wait_one(gsem)

      @pl.loop(0, TOK, unroll=4)
      def _row(r):
        for c in range(D_MODEL // LANES):
          sl = pl.ds(c * LANES, LANES)
          rows_v[bi, r, sl] = rows_v[bi, r, sl] * SCALE

      pltpu.async_copy(rows_v.at[bi], out_hbm.at[j, wid], psem)

      jn = j + HALF

      @pl.when(jn < s)
      def _():
        @pl.when(j >= HALF)
        def _():
          wait_one(psem)
        fire_gather(jn)

    @pl.loop(0, NBUF)
    def _drain(_):
      wait_one(psem)

  return sc_gather


SB = 8


def _tc_relayout_body(in_ref, out_ref):
  rows = lax.broadcasted_iota(jnp.int32, (D_MODEL, D_MODEL), 0)
  cols = lax.broadcasted_iota(jnp.int32, (D_MODEL, D_MODEL), 1)
  eye = (rows == cols).astype(jnp.float32)
  for si in range(SB):
    x = in_ref[si, 0]
    u = x[:, :D_MODEL]
    v = x[:, D_MODEL:]
    ut = lax.dot_general(u, eye, (((0,), (0,)), ((), ())),
                         preferred_element_type=jnp.float32)
    vt = lax.dot_general(v, eye, (((0,), (0,)), ((), ())),
                         preferred_element_type=jnp.float32)
    out = jnp.concatenate([ut, vt], axis=1)
    out_ref[si, :, 0] = out.reshape(8, 8, TOK)


def _make_tc_relayout(s: int):
  return pl.pallas_call(
      _tc_relayout_body,
      grid=(s // SB, NW),
      in_specs=[pl.BlockSpec((SB, 1, D_MODEL, TOK),
                             lambda sb, bt: (sb, bt, 0, 0))],
      out_specs=pl.BlockSpec((SB, 8, 1, 8, TOK),
                             lambda sb, bt: (sb, 0, bt, 0, 0)),
      out_shape=jax.ShapeDtypeStruct((s, 8, NW, 8, TOK), jnp.float32),
  )


def kernel(x, table):
  b, s = x.shape
  idx3 = x.reshape(NW, TOK, s).transpose(0, 2, 1).astype(jnp.int32)
  idx3 = idx3[:, :, _TOKEN_OF_ROW]
  t = _make_sc_gather(s)(idx3, table)
  t2 = t.reshape(s, NW, D_MODEL, TOK)
  out5 = _make_tc_relayout(s)(t2)
  return jnp.transpose(out5, (2, 4, 0, 1, 3)).reshape(b, s, D_MODEL)


# speedup vs baseline: 18.1590x; 1.7298x over previous
import functools

import jax
import jax.numpy as jnp
from jax import lax
from jax.experimental import pallas as pl
from jax.experimental.pallas import tpu as pltpu
from jax.experimental.pallas import tpu_sc as plsc

D_MODEL = 64
SCALE = 8.0

_INFO = plsc.get_sparse_core_info()
NC = _INFO.num_cores
NS = _INFO.num_subcores
NW = NC * NS
LANES = _INFO.num_lanes

CHUNK = 128
NBUF = 8
HALF = NBUF // 2


def _make_kernel(n_idx: int):
  assert n_idx % (NW * CHUNK) == 0
  per_w = n_idx // NW
  n_chunks = per_w // CHUNK
  assert n_chunks > NBUF

  mesh = plsc.VectorSubcoreMesh(core_axis_name="c", subcore_axis_name="s")

  @functools.partial(
      pl.kernel,
      out_type=jax.ShapeDtypeStruct((n_idx, D_MODEL), jnp.float32),
      mesh=mesh,
      scratch_types=[
          pltpu.VMEM((n_chunks, CHUNK), jnp.int32),
          pltpu.VMEM((NBUF, CHUNK, D_MODEL), jnp.float32),
          pltpu.SemaphoreType.DMA,
          pltpu.SemaphoreType.DMA,
      ],
      compiler_params=pltpu.CompilerParams(use_tc_tiling_on_sc=False),
  )
  def emb_kernel(idx_hbm, table_hbm, out_hbm, idx_v, rows_v, gsem, psem):
    wid = lax.axis_index("s") * NC + lax.axis_index("c")
    base = wid * per_w
    pltpu.sync_copy(idx_hbm.at[wid], idx_v)

    def fire_gather(j):
      pltpu.async_copy(table_hbm.at[idx_v.at[j]], rows_v.at[j % NBUF], gsem)

    def wait_one(sem):
      pltpu.make_async_copy(rows_v.at[0], out_hbm.at[pl.ds(0, CHUNK)],
                            sem).wait()

    for j in range(HALF):
      fire_gather(j)

    @pl.loop(0, n_chunks)
    def _chunk(j):
      bi = j % NBUF
      wait_one(gsem)

      @pl.loop(0, CHUNK, unroll=4)
      def _row(r):
        for c in range(D_MODEL // LANES):
          sl = pl.ds(c * LANES, LANES)
          rows_v[bi, r, sl] = rows_v[bi, r, sl] * SCALE

      pltpu.async_copy(
          rows_v.at[bi], out_hbm.at[pl.ds(base + j * CHUNK, CHUNK)], psem
      )

      jn = j + HALF

      @pl.when(jn < n_chunks)
      def _():
        @pl.when(j >= HALF)
        def _():
          wait_one(psem)
        fire_gather(jn)

    @pl.loop(0, NBUF)
    def _drain(_):
      wait_one(psem)

  return emb_kernel


def kernel(x, table):
  b, s = x.shape
  n_idx = b * s
  idx = x.reshape(NW, n_idx // (NW * CHUNK), CHUNK).astype(jnp.int32)
  out = _make_kernel(n_idx)(idx, table)
  return out.reshape(b, s, D_MODEL)
